# baseline (speedup 1.0000x reference)
import functools

import jax
import jax.numpy as jnp
from jax import lax
from jax.experimental import pallas as pl
from jax.experimental.pallas import tpu as pltpu
from jax.experimental.pallas import tpu_sc as plsc

NQ = 2048
NK = 2048
NH = 16
NBIAS = NQ + NK + 1
BQ = 1024
LW = NK + BQ
RB_PAD = 4352
SPAN = RB_PAD // 2
NGRP = SPAN // 16


@functools.partial(
    pl.kernel,
    out_type=jax.ShapeDtypeStruct((NH, 1, RB_PAD), jnp.float32),
    mesh=plsc.VectorSubcoreMesh(core_axis_name="c", subcore_axis_name="s"),
    compiler_params=pltpu.CompilerParams(needs_layout_passes=False),
    scratch_types=[
        pltpu.VMEM((SPAN * NH,), jnp.float32),
        pltpu.VMEM((SPAN,), jnp.float32),
        pltpu.SemaphoreType.DMA,
    ],
)
def _sc_relayout(bias_hbm, rb_hbm, bias_v, out_v, sem):
    h = lax.axis_index("s")
    half = lax.axis_index("c")
    t_base = pl.multiple_of(half * SPAN, 128)
    row_lo = pl.multiple_of((NQ + NK - SPAN) * (1 - half), 8)
    pltpu.async_copy(
        bias_hbm.at[pl.ds(row_lo * NH, SPAN * NH)], bias_v, sem
    ).wait()

    def body(g, carry):
        t0 = t_base + g * 16
        rows = (NQ + NK) - row_lo - t0 - lax.iota(jnp.int32, 16)
        mask = (rows >= 0) & (rows < SPAN)
        rows_c = jnp.where(mask, rows, 0)
        v = plsc.load_gather(bias_v, [rows_c * NH + h], mask=mask)
        out_v[pl.ds(g * 16, 16)] = jnp.where(mask, v, 0.0)
        return carry

    lax.fori_loop(0, NGRP, body, 0)
    pltpu.sync_copy(out_v, rb_hbm.at[h, 0, pl.ds(t_base, SPAN)])


def _tc_body(rb_ref, qk_ref, out_ref):
    qi = pl.program_id(1)
    base = (NQ - BQ) - qi * BQ
    w = rb_ref[0, 0, pl.ds(base, LW)]
    win = jnp.broadcast_to(w[None, :], (BQ, LW))
    rolled = pltpu.roll(win, LW - BQ, axis=1, stride=1, stride_axis=0)
    out_ref[0, 0] = qk_ref[0, 0] + rolled[:, :NK]


def kernel(qk, bias):
    rb = _sc_relayout(bias.reshape(-1))
    return pl.pallas_call(
        _tc_body,
        grid=(NH, NQ // BQ),
        in_specs=[
            pl.BlockSpec((1, 1, RB_PAD), lambda h, qi: (h, 0, 0)),
            pl.BlockSpec((1, 1, BQ, NK), lambda h, qi: (0, h, qi, 0)),
        ],
        out_specs=pl.BlockSpec((1, 1, BQ, NK), lambda h, qi: (0, h, qi, 0)),
        out_shape=jax.ShapeDtypeStruct((1, NH, NQ, NK), jnp.float32),
    )(rb, qk)
